# baseline (speedup 1.0000x reference)
import numpy as np
import jax
import jax.numpy as jnp
from jax import lax
from jax.experimental import pallas as pl
from jax.experimental.pallas import tpu as pltpu
from jax.experimental.pallas import tpu_sc as plsc

_VOCAB = 1000000
_EMBED = 64
_SEQ = 200
_BATCH = 4096

_NC = 2
_NS = 16
_NW = _NC * _NS
_BW = _BATCH // _NW
_NBUF = 2
_SCALE = float(np.sqrt(_EMBED))


def _positional_encoding():
    depth_h = _EMBED / 2
    positions = np.arange(_SEQ)[:, np.newaxis]
    depths = np.arange(depth_h)[np.newaxis, :] / depth_h
    angle_rates = 1 / 10000 ** depths
    angle_rads = positions * angle_rates
    pos = np.concatenate([np.sin(angle_rads), np.cos(angle_rads)], axis=-1)
    return pos.astype(np.float32)


_POS = _positional_encoding()


def _body(x_hbm, pos_hbm, table_hbm, out_hbm, idx_v, pos_v, grows, tbuf,
          gsem, ssem):
    c = lax.axis_index("c")
    s = lax.axis_index("s")
    wid = s * _NC + c
    b0 = wid * _BW
    pltpu.sync_copy(x_hbm.at[wid], idx_v)
    pltpu.sync_copy(pos_hbm, pos_v)
    biota = lax.iota(jnp.int32, 16)

    def gather(p, slot):
        pltpu.async_copy(table_hbm.at[idx_v.at[p]], grows.at[slot],
                         gsem.at[slot])

    def store(p, slot):
        return pltpu.make_async_copy(
            tbuf.at[slot], out_hbm.at[p, :, pl.ds(b0, _BW)], ssem.at[slot])

    for p0 in range(_NBUF):
        gather(p0, p0)

    def step(p, carry):
        slot = lax.rem(p, _NBUF)

        @pl.when(p >= _NBUF)
        def _():
            store(p - _NBUF, slot).wait()

        pltpu.make_async_copy(table_hbm.at[idx_v.at[p]], grows.at[slot],
                              gsem.at[slot]).wait()

        pvec = jnp.full((16,), 0, jnp.int32) + p

        def col(e, carry2):
            evec = jnp.full((16,), 0, jnp.int32) + e
            pe = plsc.load_gather(pos_v, [pvec, evec])
            for k in range(_BW // 16):
                bvec = biota + (k * 16)
                v = plsc.load_gather(grows.at[slot], [bvec, evec])
                tbuf[slot, e, pl.ds(k * 16, 16)] = v * _SCALE + pe
            return carry2

        lax.fori_loop(0, _EMBED, col, 0, unroll=4)

        store(p, slot).start()

        @pl.when(p + _NBUF < _SEQ)
        def _():
            gather(p + _NBUF, slot)

        return carry

    lax.fori_loop(0, _SEQ, step, 0)

    for r in range(_NBUF):
        p = _SEQ - _NBUF + r
        store(p, p % _NBUF).wait()


@jax.jit
def _run(x, table):
    pos = jnp.asarray(_POS)
    x_r = jnp.swapaxes(
        jnp.reshape(x.astype(jnp.int32), (_NW, _BW, _SEQ)), 1, 2)
    fn = pl.kernel(
        _body,
        out_type=jax.ShapeDtypeStruct((_SEQ, _EMBED, _BATCH), jnp.float32),
        mesh=plsc.VectorSubcoreMesh(
            core_axis_name="c", subcore_axis_name="s",
            num_cores=_NC, num_subcores=_NS,
        ),
        scratch_types=[
            pltpu.VMEM((_SEQ, _BW), jnp.int32),
            pltpu.VMEM((_SEQ, _EMBED), jnp.float32),
            pltpu.VMEM((_NBUF, _BW, _EMBED), jnp.float32),
            pltpu.VMEM((_NBUF, _EMBED, _BW), jnp.float32),
            pltpu.SemaphoreType.DMA((_NBUF,)),
            pltpu.SemaphoreType.DMA((_NBUF,)),
        ],
        compiler_params=pltpu.CompilerParams(
            use_tc_tiling_on_sc=False, needs_layout_passes=False),
    )
    out_t = fn(x_r, pos, table)
    return jnp.transpose(out_t, (2, 0, 1))


def kernel(x, table):
    return _run(x, table)


# speedup vs baseline: 1.4424x; 1.4424x over previous
import numpy as np
import jax
import jax.numpy as jnp
from jax import lax
from jax.experimental import pallas as pl
from jax.experimental.pallas import tpu as pltpu
from jax.experimental.pallas import tpu_sc as plsc

_VOCAB = 1000000
_EMBED = 64
_SEQ = 200
_BATCH = 4096

_NC = 2
_NS = 16
_NW = _NC * _NS
_BW = _BATCH // _NW
_NBUF = 2
_EPAD = 72
_SCALE = float(np.sqrt(_EMBED))


def _positional_encoding():
    depth_h = _EMBED / 2
    positions = np.arange(_SEQ)[:, np.newaxis]
    depths = np.arange(depth_h)[np.newaxis, :] / depth_h
    angle_rates = 1 / 10000 ** depths
    angle_rads = positions * angle_rates
    pos = np.concatenate([np.sin(angle_rads), np.cos(angle_rads)], axis=-1)
    return pos.astype(np.float32)


_POS = _positional_encoding()


def _body(x_hbm, pos_hbm, table_hbm, out_hbm, idx_v, pos_v, grows, tbuf,
          gsem, ssem):
    c = lax.axis_index("c")
    s = lax.axis_index("s")
    wid = s * _NC + c
    b0 = wid * _BW
    pltpu.sync_copy(x_hbm.at[wid], idx_v)
    pltpu.sync_copy(pos_hbm, pos_v)
    biota = lax.iota(jnp.int32, 16)

    def gather(p, slot):
        pltpu.async_copy(table_hbm.at[idx_v.at[p]], grows.at[slot],
                         gsem.at[slot])

    def store(p, slot):
        return pltpu.make_async_copy(
            tbuf.at[slot], out_hbm.at[p, :, pl.ds(b0, _BW)], ssem.at[slot])

    for p0 in range(_NBUF):
        gather(p0, p0)

    def step(p, carry):
        slot = lax.rem(p, _NBUF)

        @pl.when(p >= _NBUF)
        def _():
            store(p - _NBUF, slot).wait()

        pltpu.make_async_copy(table_hbm.at[idx_v.at[p]], grows.at[slot],
                              gsem.at[slot]).wait()

        pvec = jnp.full((16,), 0, jnp.int32) + p

        @plsc.parallel_loop(0, _EMBED, unroll=8)
        def col(e):
            evec = jnp.full((16,), 0, jnp.int32) + e
            pe = plsc.load_gather(pos_v, [pvec, evec])
            for k in range(_BW // 16):
                bvec = biota + (k * 16)
                v = plsc.load_gather(grows.at[slot], [bvec, evec])
                tbuf[slot, e, pl.ds(k * 16, 16)] = v * _SCALE + pe

        store(p, slot).start()

        @pl.when(p + _NBUF < _SEQ)
        def _():
            gather(p + _NBUF, slot)

        return carry

    lax.fori_loop(0, _SEQ, step, 0)

    for r in range(_NBUF):
        p = _SEQ - _NBUF + r
        store(p, p % _NBUF).wait()


@jax.jit
def _run(x, table):
    pos = jnp.asarray(_POS)
    table_p = jnp.pad(table, ((0, 0), (0, _EPAD - _EMBED)))
    x_r = jnp.swapaxes(
        jnp.reshape(x.astype(jnp.int32), (_NW, _BW, _SEQ)), 1, 2)
    fn = pl.kernel(
        _body,
        out_type=jax.ShapeDtypeStruct((_SEQ, _EMBED, _BATCH), jnp.float32),
        mesh=plsc.VectorSubcoreMesh(
            core_axis_name="c", subcore_axis_name="s",
            num_cores=_NC, num_subcores=_NS,
        ),
        scratch_types=[
            pltpu.VMEM((_SEQ, _BW), jnp.int32),
            pltpu.VMEM((_SEQ, _EMBED), jnp.float32),
            pltpu.VMEM((_NBUF, _BW, _EPAD), jnp.float32),
            pltpu.VMEM((_NBUF, _EMBED, _BW), jnp.float32),
            pltpu.SemaphoreType.DMA((_NBUF,)),
            pltpu.SemaphoreType.DMA((_NBUF,)),
        ],
        compiler_params=pltpu.CompilerParams(
            use_tc_tiling_on_sc=False, needs_layout_passes=False),
    )
    out_t = fn(x_r, pos, table_p)
    return jnp.transpose(out_t, (2, 0, 1))


def kernel(x, table):
    return _run(x, table)
